# baseline (speedup 1.0000x reference)
import functools

import jax
import jax.numpy as jnp
from jax import lax
from jax.experimental import pallas as pl
from jax.experimental.pallas import tpu as pltpu
from jax.experimental.pallas import tpu_sc as plsc

VOCAB = 1000000
EMBED = 64
BATCH = 4096
HIST = 200
CHUNK = 100
NC, NS = 2, 16
NW = NC * NS
ROWS_W = BATCH // NW
NCH_W = ROWS_W * (HIST // CHUNK)
XP_W = 8192
NBLK = 62
HALF = NBLK * XP_W
NCOLB = (VOCAB + XP_W - 1) // XP_W - 1


def _xpose_body(a_ref, b_ref, o_ref):
    eye = (lax.broadcasted_iota(jnp.int32, (EMBED, EMBED), 0) ==
           lax.broadcasted_iota(jnp.int32, (EMBED, EMBED), 1)
           ).astype(jnp.float32)
    dims = (((0,), (0,)), ((), ()))
    ya = lax.dot_general(a_ref[...], eye, dims,
                         preferred_element_type=jnp.float32)
    yb = lax.dot_general(b_ref[...], eye, dims,
                         preferred_element_type=jnp.float32)
    o_ref[...] = jnp.concatenate([ya, yb], axis=1)

    @pl.when(pl.program_id(0) == 0)
    def _():
        o_ref[0:1, 0:EMBED] = jnp.zeros((1, EMBED), jnp.float32)


def _row_table(table):
    tt = table.T
    return pl.pallas_call(
        _xpose_body,
        grid=(NBLK,),
        in_specs=[
            pl.BlockSpec((EMBED, XP_W), lambda i: (0, i)),
            pl.BlockSpec((EMBED, XP_W),
                         lambda i: (0, jnp.minimum(i + NBLK, NCOLB))),
        ],
        out_specs=pl.BlockSpec((XP_W, 128), lambda i: (i, 0)),
        out_shape=jax.ShapeDtypeStruct((HALF, 128), jnp.float32),
    )(tt, tt)


def _remap_body(x_ref, o_ref):
    t = x_ref[...]
    o_ref[...] = 2 * t - jnp.where(t >= HALF, 2 * HALF - 1, 0)


def _remap(x2):
    return pl.pallas_call(
        _remap_body,
        grid=(8,),
        in_specs=[pl.BlockSpec((1024, CHUNK), lambda i: (i, 0))],
        out_specs=pl.BlockSpec((1024, CHUNK), lambda i: (i, 0)),
        out_shape=jax.ShapeDtypeStruct((BATCH * 2, CHUNK), jnp.int32),
    )(x2)


NBUF = 6


def _sc_body(x_hbm, tab_hbm, out_hbm, idx_v, buf, out_v,
             sem0, sem1, sem2, sem3, sem4, sem5):
    wid = lax.axis_index("s") * NC + lax.axis_index("c")
    pltpu.sync_copy(x_hbm.at[pl.ds(wid * NCH_W, NCH_W)], idx_v)

    def fire(row, slot, sem):
        pltpu.async_copy(tab_hbm.at[idx_v.at[2 * row]],
                         buf.at[slot, pl.ds(0, CHUNK)], sem)
        pltpu.async_copy(tab_hbm.at[idx_v.at[2 * row + 1]],
                         buf.at[slot, pl.ds(CHUNK, CHUNK)], sem)

    def wait(slot, sem):
        pltpu.make_async_copy(tab_hbm.at[idx_v.at[0]],
                              buf.at[slot, pl.ds(0, CHUNK)], sem).wait()
        pltpu.make_async_copy(tab_hbm.at[idx_v.at[0]],
                              buf.at[slot, pl.ds(CHUNK, CHUNK)], sem).wait()

    sems = (sem0, sem1, sem2, sem3, sem4, sem5)
    for k in range(NBUF):
        fire(k, k, sems[k])

    def consume(row, k, sem):
        wait(k, sem)

        def racc(r, accs):
            return tuple(
                accs[g] + buf[k, r, pl.ds(g * 16, 16)] for g in range(4)
            )

        z = jnp.zeros((16,), jnp.float32)
        accs = lax.fori_loop(0, HIST, racc, (z, z, z, z), unroll=8)
        for g in range(4):
            out_v[row, pl.ds(g * 16, 16)] = accs[g]

        @pl.when(row + NBUF < ROWS_W)
        def _():
            fire(row + NBUF, k, sem)

    def body(bn, carry):
        for k in range(NBUF):
            consume(NBUF * bn + k, k, sems[k])
        return carry

    nfull = ROWS_W // NBUF
    lax.fori_loop(0, nfull, body, 0)
    for k in range(ROWS_W - NBUF * nfull):
        consume(NBUF * nfull + k, k, sems[k])
    pltpu.sync_copy(out_v, out_hbm.at[pl.ds(wid * ROWS_W, ROWS_W)])


def _pooled_sums(x2, row_tab):
    mesh = plsc.VectorSubcoreMesh(core_axis_name="c", subcore_axis_name="s")
    f = pl.kernel(
        _sc_body,
        out_type=jax.ShapeDtypeStruct((BATCH, EMBED), jnp.float32),
        mesh=mesh,
        scratch_types=[
            pltpu.VMEM((NCH_W, CHUNK), jnp.int32),
            pltpu.VMEM((NBUF, HIST, EMBED), jnp.float32),
            pltpu.VMEM((ROWS_W, EMBED), jnp.float32),
            pltpu.SemaphoreType.DMA,
            pltpu.SemaphoreType.DMA,
            pltpu.SemaphoreType.DMA,
            pltpu.SemaphoreType.DMA,
            pltpu.SemaphoreType.DMA,
            pltpu.SemaphoreType.DMA,
        ],
        compiler_params=pltpu.CompilerParams(use_tc_tiling_on_sc=False),
    )
    return f(x2, row_tab.reshape(2 * HALF, EMBED))


def _mlp_body(s_ref, w1_ref, b1_ref, w2_ref, b2_ref, o_ref):
    h = lax.dot_general(s_ref[...], w1_ref[...], (((1,), (1,)), ((), ())),
                        preferred_element_type=jnp.float32)
    h = jnp.maximum(h + b1_ref[...], 0.0)
    o_ref[...] = jnp.sum(h * w2_ref[...], axis=1, keepdims=True) + b2_ref[...]


def _mlp(s, W1, b1, W2, b2):
    return pl.pallas_call(
        _mlp_body,
        grid=(1,),
        in_specs=[
            pl.BlockSpec((BATCH, EMBED), lambda i: (0, 0)),
            pl.BlockSpec((128, EMBED), lambda i: (0, 0)),
            pl.BlockSpec((1, 128), lambda i: (0, 0)),
            pl.BlockSpec((1, 128), lambda i: (0, 0)),
            pl.BlockSpec((1, 1), lambda i: (0, 0)),
        ],
        out_specs=pl.BlockSpec((BATCH, 1), lambda i: (0, 0)),
        out_shape=jax.ShapeDtypeStruct((BATCH, 1), jnp.float32),
    )(s, W1, b1, W2, b2)


@jax.jit
def kernel(x, table, W1, b1, W2, b2):
    row_tab = _row_table(table)
    x2 = _remap(x.reshape(BATCH * 2, CHUNK))
    s = _pooled_sums(x2, row_tab)
    return _mlp(s, W1, b1.reshape(1, 128), W2, b2.reshape(1, 1))


# speedup vs baseline: 1.0028x; 1.0028x over previous
import functools

import jax
import jax.numpy as jnp
from jax import lax
from jax.experimental import pallas as pl
from jax.experimental.pallas import tpu as pltpu
from jax.experimental.pallas import tpu_sc as plsc

VOCAB = 1000000
EMBED = 64
BATCH = 4096
HIST = 200
CHUNK = 100
NC, NS = 2, 16
NW = NC * NS
ROWS_W = BATCH // NW
NCH_W = ROWS_W * (HIST // CHUNK)
XP_W = 8192
NBLK = 62
HALF = NBLK * XP_W
NCOLB = (VOCAB + XP_W - 1) // XP_W - 1


def _xpose_body(a_ref, b_ref, o_ref):
    eye = (lax.broadcasted_iota(jnp.int32, (EMBED, EMBED), 0) ==
           lax.broadcasted_iota(jnp.int32, (EMBED, EMBED), 1)
           ).astype(jnp.float32)
    dims = (((0,), (0,)), ((), ()))
    ya = lax.dot_general(a_ref[...], eye, dims,
                         preferred_element_type=jnp.float32)
    yb = lax.dot_general(b_ref[...], eye, dims,
                         preferred_element_type=jnp.float32)
    o_ref[...] = jnp.concatenate([ya, yb], axis=1)

    @pl.when(pl.program_id(0) == 0)
    def _():
        o_ref[0:1, 0:EMBED] = jnp.zeros((1, EMBED), jnp.float32)


def _row_table(table):
    tt = table.T
    return pl.pallas_call(
        _xpose_body,
        grid=(NBLK,),
        in_specs=[
            pl.BlockSpec((EMBED, XP_W), lambda i: (0, i)),
            pl.BlockSpec((EMBED, XP_W),
                         lambda i: (0, jnp.minimum(i + NBLK, NCOLB))),
        ],
        out_specs=pl.BlockSpec((XP_W, 128), lambda i: (i, 0)),
        out_shape=jax.ShapeDtypeStruct((HALF, 128), jnp.float32),
    )(tt, tt)


def _remap_body(x_ref, o_ref):
    t = x_ref[...]
    o_ref[...] = 2 * t - jnp.where(t >= HALF, 2 * HALF - 1, 0)


def _remap(x2):
    return pl.pallas_call(
        _remap_body,
        grid=(8,),
        in_specs=[pl.BlockSpec((1024, CHUNK), lambda i: (i, 0))],
        out_specs=pl.BlockSpec((1024, CHUNK), lambda i: (i, 0)),
        out_shape=jax.ShapeDtypeStruct((BATCH * 2, CHUNK), jnp.int32),
    )(x2)


NBUF = 4


def _sc_body(x_hbm, tab_hbm, out_hbm, idx_v, buf, out_v,
             sem0, sem1, sem2, sem3):
    wid = lax.axis_index("s") * NC + lax.axis_index("c")
    pltpu.sync_copy(x_hbm.at[pl.ds(wid * NCH_W, NCH_W)], idx_v)

    def fire(row, slot, sem):
        pltpu.async_copy(tab_hbm.at[idx_v.at[2 * row]],
                         buf.at[slot, pl.ds(0, CHUNK)], sem)
        pltpu.async_copy(tab_hbm.at[idx_v.at[2 * row + 1]],
                         buf.at[slot, pl.ds(CHUNK, CHUNK)], sem)

    def wait(slot, sem):
        pltpu.make_async_copy(tab_hbm.at[idx_v.at[0]],
                              buf.at[slot, pl.ds(0, CHUNK)], sem).wait()
        pltpu.make_async_copy(tab_hbm.at[idx_v.at[0]],
                              buf.at[slot, pl.ds(CHUNK, CHUNK)], sem).wait()

    sems = (sem0, sem1, sem2, sem3)
    for k in range(NBUF):
        fire(k, k, sems[k])

    def consume(row, k, sem):
        wait(k, sem)

        def racc(r, accs):
            return tuple(
                accs[g] + buf[k, r, pl.ds(g * 16, 16)] for g in range(4)
            )

        z = jnp.zeros((16,), jnp.float32)
        accs = lax.fori_loop(0, HIST, racc, (z, z, z, z), unroll=8)
        for g in range(4):
            out_v[row, pl.ds(g * 16, 16)] = accs[g]

        @pl.when(row + NBUF < ROWS_W)
        def _():
            fire(row + NBUF, k, sem)

    def body(bn, carry):
        for k in range(NBUF):
            consume(NBUF * bn + k, k, sems[k])
        return carry

    nfull = ROWS_W // NBUF
    lax.fori_loop(0, nfull, body, 0)
    for k in range(ROWS_W - NBUF * nfull):
        consume(NBUF * nfull + k, k, sems[k])
    pltpu.sync_copy(out_v, out_hbm.at[pl.ds(wid * ROWS_W, ROWS_W)])


def _pooled_sums(x2, row_tab):
    mesh = plsc.VectorSubcoreMesh(core_axis_name="c", subcore_axis_name="s")
    f = pl.kernel(
        _sc_body,
        out_type=jax.ShapeDtypeStruct((BATCH, EMBED), jnp.float32),
        mesh=mesh,
        scratch_types=[
            pltpu.VMEM((NCH_W, CHUNK), jnp.int32),
            pltpu.VMEM((NBUF, HIST, EMBED), jnp.float32),
            pltpu.VMEM((ROWS_W, EMBED), jnp.float32),
            pltpu.SemaphoreType.DMA,
            pltpu.SemaphoreType.DMA,
            pltpu.SemaphoreType.DMA,
            pltpu.SemaphoreType.DMA,
        ],
        compiler_params=pltpu.CompilerParams(use_tc_tiling_on_sc=False),
    )
    return f(x2, row_tab.reshape(2 * HALF, EMBED))


def _mlp_body(s_ref, w1_ref, b1_ref, w2_ref, b2_ref, o_ref):
    h = lax.dot_general(s_ref[...], w1_ref[...], (((1,), (1,)), ((), ())),
                        preferred_element_type=jnp.float32)
    h = jnp.maximum(h + b1_ref[...], 0.0)
    o_ref[...] = jnp.sum(h * w2_ref[...], axis=1, keepdims=True) + b2_ref[...]


def _mlp(s, W1, b1, W2, b2):
    return pl.pallas_call(
        _mlp_body,
        grid=(1,),
        in_specs=[
            pl.BlockSpec((BATCH, EMBED), lambda i: (0, 0)),
            pl.BlockSpec((128, EMBED), lambda i: (0, 0)),
            pl.BlockSpec((1, 128), lambda i: (0, 0)),
            pl.BlockSpec((1, 128), lambda i: (0, 0)),
            pl.BlockSpec((1, 1), lambda i: (0, 0)),
        ],
        out_specs=pl.BlockSpec((BATCH, 1), lambda i: (0, 0)),
        out_shape=jax.ShapeDtypeStruct((BATCH, 1), jnp.float32),
    )(s, W1, b1, W2, b2)


@jax.jit
def kernel(x, table, W1, b1, W2, b2):
    row_tab = _row_table(table)
    x2 = _remap(x.reshape(BATCH * 2, CHUNK))
    s = _pooled_sums(x2, row_tab)
    return _mlp(s, W1, b1.reshape(1, 128), W2, b2.reshape(1, 1))
